# baseline (speedup 1.0000x reference)
import functools

import jax
import jax.numpy as jnp
from jax import lax
from jax.experimental import pallas as pl
from jax.experimental.pallas import tpu as pltpu
from jax.experimental.pallas import tpu_sc as plsc

N = 10000
E = 160000
H = 256
HH = 128
G = 128
NTILES = 16
NPAD = 10240
ROWS_PER_TILE = NPAD // NTILES
ECHUNK = 128
EPAD = 163840
EDGES_PER_TILE = EPAD // NTILES
NCHUNK_T = EDGES_PER_TILE // ECHUNK
NSLOT = 2
NIDX = 8
OUTER = NCHUNK_T // NIDX
ROW_BLK = 1000
NBLK = N // ROW_BLK


@functools.cache
def _get_sc_agg():
    kern = functools.partial(
        pl.kernel,
        mesh=plsc.VectorSubcoreMesh(core_axis_name="c", subcore_axis_name="s"),
        out_type=jax.ShapeDtypeStruct((2, NPAD, HH), jnp.float32),
        scratch_types=(
            [pltpu.VMEM((ECHUNK,), jnp.int32)] * (2 * NIDX)
            + [pltpu.VMEM((ECHUNK, HH), jnp.float32)] * NSLOT
            + [pltpu.VMEM_SHARED((NPAD, HH), jnp.float32)]
            + [pltpu.SemaphoreType.DMA] * (NIDX + 2 * NSLOT)
        ),
    )
    return kern(_sc_agg_body)


def _sc_agg_body(src_hbm, dst_hbm, xs_hbm, zeros_hbm, out_hbm, *scr):
    srcb = list(scr[0:NIDX])
    dstb = list(scr[NIDX:2 * NIDX])
    rows = list(scr[2 * NIDX:2 * NIDX + NSLOT])
    agg_sh = scr[2 * NIDX + NSLOT]
    semi = list(scr[2 * NIDX + NSLOT + 1:2 * NIDX + NSLOT + 1 + NIDX])
    sg = list(scr[2 * NIDX + NSLOT + 1 + NIDX:2 * NIDX + NSLOT + 1 + NIDX + NSLOT])
    ss = list(scr[2 * NIDX + NSLOT + 1 + NIDX + NSLOT:])
    c = lax.axis_index("c")
    s = lax.axis_index("s")
    row0 = s * ROWS_PER_TILE
    e0 = s * EDGES_PER_TILE
    xc = xs_hbm.at[c]
    pltpu.sync_copy(zeros_hbm, agg_sh.at[pl.ds(row0, ROWS_PER_TILE)])

    def fire_idx(j, b):
        base = e0 + j * ECHUNK
        pltpu.async_copy(src_hbm.at[pl.ds(base, ECHUNK)], srcb[b], semi[b])
        pltpu.async_copy(dst_hbm.at[pl.ds(base, ECHUNK)], dstb[b], semi[b])

    def wait_idx(j, b):
        base = e0 + j * ECHUNK
        pltpu.make_async_copy(src_hbm.at[pl.ds(base, ECHUNK)], srcb[b], semi[b]).wait()
        pltpu.make_async_copy(dst_hbm.at[pl.ds(base, ECHUNK)], dstb[b], semi[b]).wait()

    def fire_gather(rb, ib):
        pltpu.async_copy(xc.at[srcb[ib]], rows[rb], sg[rb])

    def wait_gather(rb, ib):
        pltpu.make_async_copy(xc.at[srcb[ib]], rows[rb], sg[rb]).wait()

    def fire_scatter(rb, ib):
        pltpu.async_copy(rows[rb], agg_sh.at[dstb[ib]], ss[rb], add=True)

    def wait_scatter(rb, ib):
        pltpu.make_async_copy(rows[rb], agg_sh.at[dstb[ib]], ss[rb]).wait()

    for j in range(6):
        fire_idx(j, j)
    plsc.subcore_barrier()
    wait_idx(0, 0)
    fire_gather(0, 0)

    def outer(io, carry):
        for bb in range(NIDX):
            i = io * NIDX + bb
            rb = bb % NSLOT
            rb1 = (bb + 1) % NSLOT
            ib = bb
            ib1 = (bb + 1) % NIDX
            ib6 = (bb + 6) % NIDX
            ib7 = (bb + 7) % NIDX

            wait_gather(rb, ib)

            @pl.when(i >= 1)
            def _():
                wait_scatter(rb1, ib7)

            fire_scatter(rb, ib)

            @pl.when(i + 6 < NCHUNK_T)
            def _():
                fire_idx(i + 6, ib6)

            @pl.when(i + 1 < NCHUNK_T)
            def _():
                wait_idx(i + 1, ib1)
                fire_gather(rb1, ib1)
        return carry

    lax.fori_loop(0, OUTER, outer, 0)
    wait_scatter((NCHUNK_T - 1) % NSLOT, (NCHUNK_T - 1) % NIDX)
    plsc.subcore_barrier()
    pltpu.sync_copy(agg_sh.at[pl.ds(row0, ROWS_PER_TILE)],
                    out_hbm.at[c, pl.ds(row0, ROWS_PER_TILE)])


def _embed_body(x_ref, W_ref, b_ref, y_ref):
    y = jnp.dot(x_ref[...], W_ref[...], preferred_element_type=jnp.float32,
                precision=lax.Precision.HIGHEST)
    y = y + b_ref[...]
    y_ref[0] = y[:, :HH]
    y_ref[1] = y[:, HH:]


def _embed(x, W, b):
    F = x.shape[1]
    return pl.pallas_call(
        _embed_body,
        grid=(NBLK,),
        in_specs=[
            pl.BlockSpec((ROW_BLK, F), lambda i: (i, 0)),
            pl.BlockSpec((F, H), lambda i: (0, 0)),
            pl.BlockSpec((1, H), lambda i: (0, 0)),
        ],
        out_specs=pl.BlockSpec((2, ROW_BLK, HH), lambda i: (0, i, 0)),
        out_shape=jax.ShapeDtypeStruct((2, N, HH), jnp.float32),
    )(x, W, b)


def _mlp_body(x_ref, a_ref, W1_ref, b1_ref, W2_ref, b2_ref, y_ref):
    x = jnp.concatenate([x_ref[0], x_ref[1]], axis=1)
    h = x + jnp.concatenate([a_ref[0], a_ref[1]], axis=1)
    t = jnp.dot(h, W1_ref[...], preferred_element_type=jnp.float32,
                precision=lax.Precision.HIGHEST) + b1_ref[...]
    t = jnp.maximum(t, 0.0)
    y = x + jnp.dot(t, W2_ref[...], preferred_element_type=jnp.float32,
                precision=lax.Precision.HIGHEST) + b2_ref[...]
    y_ref[0] = y[:, :HH]
    y_ref[1] = y[:, HH:]


def _mlp(xs, agg, W1, b1, W2, b2):
    return pl.pallas_call(
        _mlp_body,
        grid=(NBLK,),
        in_specs=[
            pl.BlockSpec((2, ROW_BLK, HH), lambda i: (0, i, 0)),
            pl.BlockSpec((2, ROW_BLK, HH), lambda i: (0, i, 0)),
            pl.BlockSpec((H, H), lambda i: (0, 0)),
            pl.BlockSpec((1, H), lambda i: (0, 0)),
            pl.BlockSpec((H, H), lambda i: (0, 0)),
            pl.BlockSpec((1, H), lambda i: (0, 0)),
        ],
        out_specs=pl.BlockSpec((2, ROW_BLK, HH), lambda i: (0, i, 0)),
        out_shape=jax.ShapeDtypeStruct((2, N, HH), jnp.float32),
    )(xs, agg, W1, b1, W2, b2)


def _pool_body(lh_ref, ph_ref, lb_ref, pb_ref,
               W1a_ref, W1b_ref, b1_ref, W2_ref, b2_ref, out_ref,
               accL, accP, cntL, cntP):
    i = pl.program_id(0)

    @pl.when(i == 0)
    def _():
        accL[...] = jnp.zeros_like(accL)
        accP[...] = jnp.zeros_like(accP)
        cntL[...] = jnp.zeros_like(cntL)
        cntP[...] = jnp.zeros_like(cntP)

    seg_iota = lax.broadcasted_iota(jnp.int32, (G, ROW_BLK), 0)

    lb = lb_ref[0, 0, :]
    onehotL = (lb[None, :] == seg_iota).astype(jnp.float32)
    hl = jnp.concatenate([lh_ref[0], lh_ref[1]], axis=1)
    accL[...] += jnp.dot(onehotL, hl, preferred_element_type=jnp.float32,
                precision=lax.Precision.HIGHEST)
    cntL[...] += jnp.sum(onehotL, axis=1, keepdims=True)

    pb = pb_ref[0, 0, :]
    onehotP = (pb[None, :] == seg_iota).astype(jnp.float32)
    hp = jnp.concatenate([ph_ref[0], ph_ref[1]], axis=1)
    accP[...] += jnp.dot(onehotP, hp, preferred_element_type=jnp.float32,
                precision=lax.Precision.HIGHEST)
    cntP[...] += jnp.sum(onehotP, axis=1, keepdims=True)

    @pl.when(i == NBLK - 1)
    def _():
        poolL = accL[...] / jnp.maximum(cntL[...], 1.0)
        poolP = accP[...] / jnp.maximum(cntP[...], 1.0)
        t = jnp.dot(poolL, W1a_ref[...], preferred_element_type=jnp.float32,
                precision=lax.Precision.HIGHEST)
        t = t + jnp.dot(poolP, W1b_ref[...], preferred_element_type=jnp.float32,
                precision=lax.Precision.HIGHEST)
        t = jnp.maximum(t + b1_ref[...], 0.0)
        out_ref[...] = (jnp.dot(t, W2_ref[...], preferred_element_type=jnp.float32,
                precision=lax.Precision.HIGHEST)
                        + b2_ref[...])


def _pool_pred(lh, ph, lb3, pb3, W1a, W1b, b1, W2p, b2p):
    return pl.pallas_call(
        _pool_body,
        grid=(NBLK,),
        in_specs=[
            pl.BlockSpec((2, ROW_BLK, HH), lambda i: (0, i, 0)),
            pl.BlockSpec((2, ROW_BLK, HH), lambda i: (0, i, 0)),
            pl.BlockSpec((1, 1, ROW_BLK), lambda i: (i, 0, 0)),
            pl.BlockSpec((1, 1, ROW_BLK), lambda i: (i, 0, 0)),
            pl.BlockSpec((H, H), lambda i: (0, 0)),
            pl.BlockSpec((H, H), lambda i: (0, 0)),
            pl.BlockSpec((1, H), lambda i: (0, 0)),
            pl.BlockSpec((H, G), lambda i: (0, 0)),
            pl.BlockSpec((1, G), lambda i: (0, 0)),
        ],
        out_specs=pl.BlockSpec((G, G), lambda i: (0, 0)),
        out_shape=jax.ShapeDtypeStruct((G, G), jnp.float32),
        scratch_shapes=[
            pltpu.VMEM((G, H), jnp.float32),
            pltpu.VMEM((G, H), jnp.float32),
            pltpu.VMEM((G, 1), jnp.float32),
            pltpu.VMEM((G, 1), jnp.float32),
        ],
    )(lh, ph, lb3, pb3, W1a, W1b, b1, W2p, b2p)


def kernel(ligand_x, protein_x, ligand_edge_index, protein_edge_index,
           ligand_batch, protein_batch,
           lig_embed_W, lig_embed_b, prot_embed_W, prot_embed_b,
           lig_W1, lig_b1, lig_W2, lig_b2,
           prot_W1, prot_b1, prot_W2, prot_b2,
           pred_W1, pred_b1, pred_W2, pred_b2):
    def _prep_edges(edge_index):
        pad = EPAD - edge_index.shape[1]
        src = jnp.concatenate([edge_index[0].astype(jnp.int32),
                               jnp.zeros((pad,), jnp.int32)])
        dst = jnp.concatenate([edge_index[1].astype(jnp.int32),
                               jnp.full((pad,), NPAD - 1, jnp.int32)])
        return src, dst

    lsrc, ldst = _prep_edges(ligand_edge_index)
    psrc, pdst = _prep_edges(protein_edge_index)
    zeros = jnp.zeros((ROWS_PER_TILE, HH), jnp.float32)

    lh = _embed(ligand_x, lig_embed_W, lig_embed_b.reshape(1, -1))
    ph = _embed(protein_x, prot_embed_W, prot_embed_b.reshape(1, -1))

    L = lig_W1.shape[0]
    for i in range(L):
        lagg = _get_sc_agg()(lsrc, ldst, lh, zeros)
        pagg = _get_sc_agg()(psrc, pdst, ph, zeros)
        lh = _mlp(lh, lagg, lig_W1[i], lig_b1[i].reshape(1, -1),
                  lig_W2[i], lig_b2[i].reshape(1, -1))
        ph = _mlp(ph, pagg, prot_W1[i], prot_b1[i].reshape(1, -1),
                  prot_W2[i], prot_b2[i].reshape(1, -1))

    lb3 = ligand_batch.astype(jnp.int32).reshape(NBLK, 1, ROW_BLK)
    pb3 = protein_batch.astype(jnp.int32).reshape(NBLK, 1, ROW_BLK)
    W1a = pred_W1[:H]
    W1b = pred_W1[H:]
    W2p = jnp.pad(pred_W2, ((0, 0), (0, G - pred_W2.shape[1])))
    b2p = jnp.pad(pred_b2, (0, G - pred_b2.shape[0])).reshape(1, -1)

    out = _pool_pred(lh, ph, lb3, pb3, W1a, W1b,
                     pred_b1.reshape(1, -1), W2p, b2p)
    return out[:, :1]


# speedup vs baseline: 2.8600x; 2.8600x over previous
import functools

import jax
import jax.numpy as jnp
from jax import lax
from jax.experimental import pallas as pl
from jax.experimental.pallas import tpu as pltpu
from jax.experimental.pallas import tpu_sc as plsc

N = 10000
E = 160000
H = 256
HH = 128
G = 128
NTILES = 16
NPAD = 10240
ROWS_PER_TILE = NPAD // NTILES
EDGES_PER_TILE = E // NTILES
ECHUNK = 80
NCHUNK_T = EDGES_PER_TILE // ECHUNK
NSLOT = 4
NIDX = 8
OUTER = 16
ROW_BLK = 1000
NBLK = N // ROW_BLK


@functools.cache
def _get_sc_agg():
    kern = functools.partial(
        pl.kernel,
        mesh=plsc.VectorSubcoreMesh(core_axis_name="c", subcore_axis_name="s"),
        out_type=[jax.ShapeDtypeStruct((2, NPAD, HH), jnp.float32),
                  jax.ShapeDtypeStruct((2, NPAD, HH), jnp.float32)],
        scratch_types=(
            [pltpu.VMEM((ECHUNK,), jnp.int32)] * (2 * NIDX)
            + [pltpu.VMEM((ECHUNK, HH), jnp.float32)] * NSLOT
            + [pltpu.VMEM_SHARED((NPAD, HH), jnp.float32)]
            + [pltpu.SemaphoreType.DMA] * (NIDX + 2 * NSLOT)
        ),
    )
    return kern(_sc_agg_body)


def _sc_agg_body(lsrc_hbm, ldst_hbm, psrc_hbm, pdst_hbm, lxs_hbm, pxs_hbm,
                 zeros_hbm, lout_hbm, pout_hbm, *scr):
    srcb = list(scr[0:NIDX])
    dstb = list(scr[NIDX:2 * NIDX])
    rows = list(scr[2 * NIDX:2 * NIDX + NSLOT])
    agg_sh = scr[2 * NIDX + NSLOT]
    semi = list(scr[2 * NIDX + NSLOT + 1:2 * NIDX + NSLOT + 1 + NIDX])
    sg = list(scr[2 * NIDX + NSLOT + 1 + NIDX:2 * NIDX + NSLOT + 1 + NIDX + NSLOT])
    ss = list(scr[2 * NIDX + NSLOT + 1 + NIDX + NSLOT:])
    c = lax.axis_index("c")
    s = lax.axis_index("s")
    row0 = s * ROWS_PER_TILE
    e0 = s * EDGES_PER_TILE

    def run_graph(src_hbm, dst_hbm, xs_hbm, out_hbm):
        xc = xs_hbm.at[c]
        pltpu.sync_copy(zeros_hbm, agg_sh.at[pl.ds(row0, ROWS_PER_TILE)])

        def fire_idx(j, b):
            base = e0 + j * ECHUNK
            pltpu.async_copy(src_hbm.at[pl.ds(base, ECHUNK)], srcb[b], semi[b])
            pltpu.async_copy(dst_hbm.at[pl.ds(base, ECHUNK)], dstb[b], semi[b])

        def wait_idx(j, b):
            base = e0 + j * ECHUNK
            pltpu.make_async_copy(src_hbm.at[pl.ds(base, ECHUNK)], srcb[b], semi[b]).wait()
            pltpu.make_async_copy(dst_hbm.at[pl.ds(base, ECHUNK)], dstb[b], semi[b]).wait()

        def fire_gather(rb, ib):
            pltpu.async_copy(xc.at[srcb[ib]], rows[rb], sg[rb])

        def wait_gather(rb, ib):
            pltpu.make_async_copy(xc.at[srcb[ib]], rows[rb], sg[rb]).wait()

        def fire_scatter(rb, ib):
            pltpu.async_copy(rows[rb], agg_sh.at[dstb[ib]], ss[rb], add=True)

        def wait_scatter(rb, ib):
            pltpu.make_async_copy(rows[rb], agg_sh.at[dstb[ib]], ss[rb]).wait()

        for j in range(6):
            fire_idx(j, j)
        plsc.subcore_barrier()
        wait_idx(0, 0)
        fire_gather(0, 0)
        wait_idx(1, 1)
        fire_gather(1, 1)

        def outer(io, carry):
            for bb in range(NIDX):
                i = io * NIDX + bb
                rb = bb % NSLOT
                rb1 = (bb + 3) % NSLOT
                rb2 = (bb + 2) % NSLOT
                ib = bb
                ib2 = (bb + 2) % NIDX
                ib6 = (bb + 6) % NIDX
                ib7 = (bb + 7) % NIDX

                @pl.when(i < NCHUNK_T)
                def _():
                    wait_gather(rb, ib)

                @pl.when(jnp.logical_and(i >= 1, i - 1 < NCHUNK_T))
                def _():
                    wait_scatter(rb1, ib7)

                @pl.when(i < NCHUNK_T)
                def _():
                    fire_scatter(rb, ib)

                @pl.when(i + 6 < NCHUNK_T)
                def _():
                    fire_idx(i + 6, ib6)

                @pl.when(i + 2 < NCHUNK_T)
                def _():
                    wait_idx(i + 2, ib2)
                    fire_gather(rb2, ib2)
            return carry

        lax.fori_loop(0, OUTER, outer, 0)
        plsc.subcore_barrier()
        pltpu.sync_copy(agg_sh.at[pl.ds(row0, ROWS_PER_TILE)],
                        out_hbm.at[c, pl.ds(row0, ROWS_PER_TILE)])

    run_graph(lsrc_hbm, ldst_hbm, lxs_hbm, lout_hbm)
    run_graph(psrc_hbm, pdst_hbm, pxs_hbm, pout_hbm)


def _embed_body(x_ref, W_ref, b_ref, y_ref):
    y = jnp.dot(x_ref[...], W_ref[...], preferred_element_type=jnp.float32)
    y = y + b_ref[...]
    y_ref[0] = y[:, :HH]
    y_ref[1] = y[:, HH:]


def _embed(x, W, b):
    F = x.shape[1]
    return pl.pallas_call(
        _embed_body,
        grid=(NBLK,),
        in_specs=[
            pl.BlockSpec((ROW_BLK, F), lambda i: (i, 0)),
            pl.BlockSpec((F, H), lambda i: (0, 0)),
            pl.BlockSpec((1, H), lambda i: (0, 0)),
        ],
        out_specs=pl.BlockSpec((2, ROW_BLK, HH), lambda i: (0, i, 0)),
        out_shape=jax.ShapeDtypeStruct((2, N, HH), jnp.float32),
    )(x, W, b)


def _mlp_body(x_ref, a_ref, W1_ref, b1_ref, W2_ref, b2_ref, y_ref):
    x = jnp.concatenate([x_ref[0], x_ref[1]], axis=1)
    h = x + jnp.concatenate([a_ref[0], a_ref[1]], axis=1)
    t = jnp.dot(h, W1_ref[...], preferred_element_type=jnp.float32) + b1_ref[...]
    t = jnp.maximum(t, 0.0)
    y = x + jnp.dot(t, W2_ref[...], preferred_element_type=jnp.float32) + b2_ref[...]
    y_ref[0] = y[:, :HH]
    y_ref[1] = y[:, HH:]


def _mlp(xs, agg, W1, b1, W2, b2):
    return pl.pallas_call(
        _mlp_body,
        grid=(NBLK,),
        in_specs=[
            pl.BlockSpec((2, ROW_BLK, HH), lambda i: (0, i, 0)),
            pl.BlockSpec((2, ROW_BLK, HH), lambda i: (0, i, 0)),
            pl.BlockSpec((H, H), lambda i: (0, 0)),
            pl.BlockSpec((1, H), lambda i: (0, 0)),
            pl.BlockSpec((H, H), lambda i: (0, 0)),
            pl.BlockSpec((1, H), lambda i: (0, 0)),
        ],
        out_specs=pl.BlockSpec((2, ROW_BLK, HH), lambda i: (0, i, 0)),
        out_shape=jax.ShapeDtypeStruct((2, N, HH), jnp.float32),
    )(xs, agg, W1, b1, W2, b2)


def _pool_body(lh_ref, ph_ref, lb_ref, pb_ref,
               W1a_ref, W1b_ref, b1_ref, W2_ref, b2_ref, out_ref,
               accL, accP, cntL, cntP):
    i = pl.program_id(0)

    @pl.when(i == 0)
    def _():
        accL[...] = jnp.zeros_like(accL)
        accP[...] = jnp.zeros_like(accP)
        cntL[...] = jnp.zeros_like(cntL)
        cntP[...] = jnp.zeros_like(cntP)

    seg_iota = lax.broadcasted_iota(jnp.int32, (G, ROW_BLK), 0)

    lb = lb_ref[0, 0, :]
    onehotL = (lb[None, :] == seg_iota).astype(jnp.float32)
    hl = jnp.concatenate([lh_ref[0], lh_ref[1]], axis=1)
    accL[...] += jnp.dot(onehotL, hl, preferred_element_type=jnp.float32)
    cntL[...] += jnp.sum(onehotL, axis=1, keepdims=True)

    pb = pb_ref[0, 0, :]
    onehotP = (pb[None, :] == seg_iota).astype(jnp.float32)
    hp = jnp.concatenate([ph_ref[0], ph_ref[1]], axis=1)
    accP[...] += jnp.dot(onehotP, hp, preferred_element_type=jnp.float32)
    cntP[...] += jnp.sum(onehotP, axis=1, keepdims=True)

    @pl.when(i == NBLK - 1)
    def _():
        poolL = accL[...] / jnp.maximum(cntL[...], 1.0)
        poolP = accP[...] / jnp.maximum(cntP[...], 1.0)
        t = jnp.dot(poolL, W1a_ref[...], preferred_element_type=jnp.float32)
        t = t + jnp.dot(poolP, W1b_ref[...], preferred_element_type=jnp.float32)
        t = jnp.maximum(t + b1_ref[...], 0.0)
        out_ref[...] = (jnp.dot(t, W2_ref[...], preferred_element_type=jnp.float32)
                        + b2_ref[...])


def _pool_pred(lh, ph, lb3, pb3, W1a, W1b, b1, W2p, b2p):
    return pl.pallas_call(
        _pool_body,
        grid=(NBLK,),
        in_specs=[
            pl.BlockSpec((2, ROW_BLK, HH), lambda i: (0, i, 0)),
            pl.BlockSpec((2, ROW_BLK, HH), lambda i: (0, i, 0)),
            pl.BlockSpec((1, 1, ROW_BLK), lambda i: (i, 0, 0)),
            pl.BlockSpec((1, 1, ROW_BLK), lambda i: (i, 0, 0)),
            pl.BlockSpec((H, H), lambda i: (0, 0)),
            pl.BlockSpec((H, H), lambda i: (0, 0)),
            pl.BlockSpec((1, H), lambda i: (0, 0)),
            pl.BlockSpec((H, G), lambda i: (0, 0)),
            pl.BlockSpec((1, G), lambda i: (0, 0)),
        ],
        out_specs=pl.BlockSpec((G, G), lambda i: (0, 0)),
        out_shape=jax.ShapeDtypeStruct((G, G), jnp.float32),
        scratch_shapes=[
            pltpu.VMEM((G, H), jnp.float32),
            pltpu.VMEM((G, H), jnp.float32),
            pltpu.VMEM((G, 1), jnp.float32),
            pltpu.VMEM((G, 1), jnp.float32),
        ],
    )(lh, ph, lb3, pb3, W1a, W1b, b1, W2p, b2p)


def kernel(ligand_x, protein_x, ligand_edge_index, protein_edge_index,
           ligand_batch, protein_batch,
           lig_embed_W, lig_embed_b, prot_embed_W, prot_embed_b,
           lig_W1, lig_b1, lig_W2, lig_b2,
           prot_W1, prot_b1, prot_W2, prot_b2,
           pred_W1, pred_b1, pred_W2, pred_b2):
    lsrc = ligand_edge_index[0].astype(jnp.int32)
    ldst = ligand_edge_index[1].astype(jnp.int32)
    psrc = protein_edge_index[0].astype(jnp.int32)
    pdst = protein_edge_index[1].astype(jnp.int32)
    zeros = jnp.zeros((ROWS_PER_TILE, HH), jnp.float32)

    lh = _embed(ligand_x, lig_embed_W, lig_embed_b.reshape(1, -1))
    ph = _embed(protein_x, prot_embed_W, prot_embed_b.reshape(1, -1))

    L = lig_W1.shape[0]
    for i in range(L):
        lagg, pagg = _get_sc_agg()(lsrc, ldst, psrc, pdst, lh, ph, zeros)
        lh = _mlp(lh, lagg, lig_W1[i], lig_b1[i].reshape(1, -1),
                  lig_W2[i], lig_b2[i].reshape(1, -1))
        ph = _mlp(ph, pagg, prot_W1[i], prot_b1[i].reshape(1, -1),
                  prot_W2[i], prot_b2[i].reshape(1, -1))

    lb3 = ligand_batch.astype(jnp.int32).reshape(NBLK, 1, ROW_BLK)
    pb3 = protein_batch.astype(jnp.int32).reshape(NBLK, 1, ROW_BLK)
    W1a = pred_W1[:H]
    W1b = pred_W1[H:]
    W2p = jnp.pad(pred_W2, ((0, 0), (0, G - pred_W2.shape[1])))
    b2p = jnp.pad(pred_b2, (0, G - pred_b2.shape[0])).reshape(1, -1)

    out = _pool_pred(lh, ph, lb3, pb3, W1a, W1b,
                     pred_b1.reshape(1, -1), W2p, b2p)
    return out[:, :1]


# speedup vs baseline: 3.0165x; 1.0548x over previous
import functools

import jax
import jax.numpy as jnp
from jax import lax
from jax.experimental import pallas as pl
from jax.experimental.pallas import tpu as pltpu
from jax.experimental.pallas import tpu_sc as plsc

N = 10000
E = 160000
H = 256
HH = 128
G = 128
NTILES = 16
NPAD = 10240
ROWS_PER_TILE = NPAD // NTILES
EDGES_PER_TILE = E // NTILES
ECHUNK = 80
NCHUNK_T = EDGES_PER_TILE // ECHUNK
NSLOT = 4
NIDX = 8
OUTER = 16
ROW_BLK = 1000
NBLK = N // ROW_BLK


@functools.cache
def _get_sc_agg():
    kern = functools.partial(
        pl.kernel,
        mesh=plsc.VectorSubcoreMesh(core_axis_name="c", subcore_axis_name="s"),
        out_type=jax.ShapeDtypeStruct((2, NPAD, HH), jnp.float32),
        scratch_types=(
            [pltpu.VMEM((ECHUNK,), jnp.int32)] * (2 * NIDX)
            + [pltpu.VMEM((ECHUNK, HH), jnp.float32)] * NSLOT
            + [pltpu.VMEM_SHARED((NPAD, HH), jnp.float32)]
            + [pltpu.SemaphoreType.DMA] * (NIDX + 2 * NSLOT)
        ),
    )
    return kern(_sc_agg_body)


def _sc_agg_body(src_hbm, dst_hbm, xs_hbm, zeros_hbm, out_hbm, *scr):
    srcb = list(scr[0:NIDX])
    dstb = list(scr[NIDX:2 * NIDX])
    rows = list(scr[2 * NIDX:2 * NIDX + NSLOT])
    agg_sh = scr[2 * NIDX + NSLOT]
    semi = list(scr[2 * NIDX + NSLOT + 1:2 * NIDX + NSLOT + 1 + NIDX])
    sg = list(scr[2 * NIDX + NSLOT + 1 + NIDX:2 * NIDX + NSLOT + 1 + NIDX + NSLOT])
    ss = list(scr[2 * NIDX + NSLOT + 1 + NIDX + NSLOT:])
    c = lax.axis_index("c")
    s = lax.axis_index("s")
    row0 = s * ROWS_PER_TILE
    e0 = s * EDGES_PER_TILE

    def run_graph(src_hbm, dst_hbm, xs_hbm, out_hbm):
        xc = xs_hbm.at[c]
        pltpu.sync_copy(zeros_hbm, agg_sh.at[pl.ds(row0, ROWS_PER_TILE)])

        def fire_idx(j, b):
            base = e0 + j * ECHUNK
            pltpu.async_copy(src_hbm.at[pl.ds(base, ECHUNK)], srcb[b], semi[b])
            pltpu.async_copy(dst_hbm.at[pl.ds(base, ECHUNK)], dstb[b], semi[b])

        def wait_idx(j, b):
            base = e0 + j * ECHUNK
            pltpu.make_async_copy(src_hbm.at[pl.ds(base, ECHUNK)], srcb[b], semi[b]).wait()
            pltpu.make_async_copy(dst_hbm.at[pl.ds(base, ECHUNK)], dstb[b], semi[b]).wait()

        def fire_gather(rb, ib):
            pltpu.async_copy(xc.at[srcb[ib]], rows[rb], sg[rb])

        def wait_gather(rb, ib):
            pltpu.make_async_copy(xc.at[srcb[ib]], rows[rb], sg[rb]).wait()

        def fire_scatter(rb, ib):
            pltpu.async_copy(rows[rb], agg_sh.at[dstb[ib]], ss[rb], add=True)

        def wait_scatter(rb, ib):
            pltpu.make_async_copy(rows[rb], agg_sh.at[dstb[ib]], ss[rb]).wait()

        for j in range(6):
            fire_idx(j, j)
        plsc.subcore_barrier()
        wait_idx(0, 0)
        fire_gather(0, 0)
        wait_idx(1, 1)
        fire_gather(1, 1)

        def outer(io, carry):
            for bb in range(NIDX):
                i = io * NIDX + bb
                rb = bb % NSLOT
                rb1 = (bb + 3) % NSLOT
                rb2 = (bb + 2) % NSLOT
                ib = bb
                ib2 = (bb + 2) % NIDX
                ib6 = (bb + 6) % NIDX
                ib7 = (bb + 7) % NIDX

                @pl.when(i < NCHUNK_T)
                def _():
                    wait_gather(rb, ib)

                @pl.when(jnp.logical_and(i >= 1, i - 1 < NCHUNK_T))
                def _():
                    wait_scatter(rb1, ib7)

                @pl.when(i < NCHUNK_T)
                def _():
                    fire_scatter(rb, ib)

                @pl.when(i + 6 < NCHUNK_T)
                def _():
                    fire_idx(i + 6, ib6)

                @pl.when(i + 2 < NCHUNK_T)
                def _():
                    wait_idx(i + 2, ib2)
                    fire_gather(rb2, ib2)
            return carry

        lax.fori_loop(0, OUTER, outer, 0)
        plsc.subcore_barrier()
        pltpu.sync_copy(agg_sh.at[pl.ds(row0, ROWS_PER_TILE)],
                        out_hbm.at[c, pl.ds(row0, ROWS_PER_TILE)])

    run_graph(src_hbm, dst_hbm, xs_hbm, out_hbm)


def _embed_body(x_ref, W_ref, b_ref, y_ref):
    y = jnp.dot(x_ref[...], W_ref[...], preferred_element_type=jnp.float32)
    y = y + b_ref[...]
    y_ref[0] = y[:, :HH]
    y_ref[1] = y[:, HH:]


def _embed(x, W, b):
    F = x.shape[1]
    return pl.pallas_call(
        _embed_body,
        grid=(NBLK,),
        in_specs=[
            pl.BlockSpec((ROW_BLK, F), lambda i: (i, 0)),
            pl.BlockSpec((F, H), lambda i: (0, 0)),
            pl.BlockSpec((1, H), lambda i: (0, 0)),
        ],
        out_specs=pl.BlockSpec((2, ROW_BLK, HH), lambda i: (0, i, 0)),
        out_shape=jax.ShapeDtypeStruct((2, N, HH), jnp.float32),
    )(x, W, b)


def _mlp_body(x_ref, a_ref, W1_ref, b1_ref, W2_ref, b2_ref, y_ref):
    x = jnp.concatenate([x_ref[0], x_ref[1]], axis=1)
    h = x + jnp.concatenate([a_ref[0], a_ref[1]], axis=1)
    t = jnp.dot(h, W1_ref[...], preferred_element_type=jnp.float32) + b1_ref[...]
    t = jnp.maximum(t, 0.0)
    y = x + jnp.dot(t, W2_ref[...], preferred_element_type=jnp.float32) + b2_ref[...]
    y_ref[0] = y[:, :HH]
    y_ref[1] = y[:, HH:]


def _mlp(xs, agg, W1, b1, W2, b2):
    return pl.pallas_call(
        _mlp_body,
        grid=(NBLK,),
        in_specs=[
            pl.BlockSpec((2, ROW_BLK, HH), lambda i: (0, i, 0)),
            pl.BlockSpec((2, ROW_BLK, HH), lambda i: (0, i, 0)),
            pl.BlockSpec((H, H), lambda i: (0, 0)),
            pl.BlockSpec((1, H), lambda i: (0, 0)),
            pl.BlockSpec((H, H), lambda i: (0, 0)),
            pl.BlockSpec((1, H), lambda i: (0, 0)),
        ],
        out_specs=pl.BlockSpec((2, ROW_BLK, HH), lambda i: (0, i, 0)),
        out_shape=jax.ShapeDtypeStruct((2, N, HH), jnp.float32),
    )(xs, agg, W1, b1, W2, b2)


def _pool_body(lh_ref, ph_ref, lb_ref, pb_ref,
               W1a_ref, W1b_ref, b1_ref, W2_ref, b2_ref, out_ref,
               accL, accP, cntL, cntP):
    i = pl.program_id(0)

    @pl.when(i == 0)
    def _():
        accL[...] = jnp.zeros_like(accL)
        accP[...] = jnp.zeros_like(accP)
        cntL[...] = jnp.zeros_like(cntL)
        cntP[...] = jnp.zeros_like(cntP)

    seg_iota = lax.broadcasted_iota(jnp.int32, (G, ROW_BLK), 0)

    lb = lb_ref[0, 0, :]
    onehotL = (lb[None, :] == seg_iota).astype(jnp.float32)
    hl = jnp.concatenate([lh_ref[0], lh_ref[1]], axis=1)
    accL[...] += jnp.dot(onehotL, hl, preferred_element_type=jnp.float32)
    cntL[...] += jnp.sum(onehotL, axis=1, keepdims=True)

    pb = pb_ref[0, 0, :]
    onehotP = (pb[None, :] == seg_iota).astype(jnp.float32)
    hp = jnp.concatenate([ph_ref[0], ph_ref[1]], axis=1)
    accP[...] += jnp.dot(onehotP, hp, preferred_element_type=jnp.float32)
    cntP[...] += jnp.sum(onehotP, axis=1, keepdims=True)

    @pl.when(i == NBLK - 1)
    def _():
        poolL = accL[...] / jnp.maximum(cntL[...], 1.0)
        poolP = accP[...] / jnp.maximum(cntP[...], 1.0)
        t = jnp.dot(poolL, W1a_ref[...], preferred_element_type=jnp.float32)
        t = t + jnp.dot(poolP, W1b_ref[...], preferred_element_type=jnp.float32)
        t = jnp.maximum(t + b1_ref[...], 0.0)
        out_ref[...] = (jnp.dot(t, W2_ref[...], preferred_element_type=jnp.float32)
                        + b2_ref[...])


def _pool_pred(lh, ph, lb3, pb3, W1a, W1b, b1, W2p, b2p):
    return pl.pallas_call(
        _pool_body,
        grid=(NBLK,),
        in_specs=[
            pl.BlockSpec((2, ROW_BLK, HH), lambda i: (0, i, 0)),
            pl.BlockSpec((2, ROW_BLK, HH), lambda i: (0, i, 0)),
            pl.BlockSpec((1, 1, ROW_BLK), lambda i: (i, 0, 0)),
            pl.BlockSpec((1, 1, ROW_BLK), lambda i: (i, 0, 0)),
            pl.BlockSpec((H, H), lambda i: (0, 0)),
            pl.BlockSpec((H, H), lambda i: (0, 0)),
            pl.BlockSpec((1, H), lambda i: (0, 0)),
            pl.BlockSpec((H, G), lambda i: (0, 0)),
            pl.BlockSpec((1, G), lambda i: (0, 0)),
        ],
        out_specs=pl.BlockSpec((G, G), lambda i: (0, 0)),
        out_shape=jax.ShapeDtypeStruct((G, G), jnp.float32),
        scratch_shapes=[
            pltpu.VMEM((G, H), jnp.float32),
            pltpu.VMEM((G, H), jnp.float32),
            pltpu.VMEM((G, 1), jnp.float32),
            pltpu.VMEM((G, 1), jnp.float32),
        ],
    )(lh, ph, lb3, pb3, W1a, W1b, b1, W2p, b2p)


def kernel(ligand_x, protein_x, ligand_edge_index, protein_edge_index,
           ligand_batch, protein_batch,
           lig_embed_W, lig_embed_b, prot_embed_W, prot_embed_b,
           lig_W1, lig_b1, lig_W2, lig_b2,
           prot_W1, prot_b1, prot_W2, prot_b2,
           pred_W1, pred_b1, pred_W2, pred_b2):
    lsrc = ligand_edge_index[0].astype(jnp.int32)
    ldst = ligand_edge_index[1].astype(jnp.int32)
    psrc = protein_edge_index[0].astype(jnp.int32)
    pdst = protein_edge_index[1].astype(jnp.int32)
    zeros = jnp.zeros((ROWS_PER_TILE, HH), jnp.float32)

    lh = _embed(ligand_x, lig_embed_W, lig_embed_b.reshape(1, -1))
    ph = _embed(protein_x, prot_embed_W, prot_embed_b.reshape(1, -1))

    L = lig_W1.shape[0]
    for i in range(L):
        lagg = _get_sc_agg()(lsrc, ldst, lh, zeros)
        pagg = _get_sc_agg()(psrc, pdst, ph, zeros)
        lh = _mlp(lh, lagg, lig_W1[i], lig_b1[i].reshape(1, -1),
                  lig_W2[i], lig_b2[i].reshape(1, -1))
        ph = _mlp(ph, pagg, prot_W1[i], prot_b1[i].reshape(1, -1),
                  prot_W2[i], prot_b2[i].reshape(1, -1))

    lb3 = ligand_batch.astype(jnp.int32).reshape(NBLK, 1, ROW_BLK)
    pb3 = protein_batch.astype(jnp.int32).reshape(NBLK, 1, ROW_BLK)
    W1a = pred_W1[:H]
    W1b = pred_W1[H:]
    W2p = jnp.pad(pred_W2, ((0, 0), (0, G - pred_W2.shape[1])))
    b2p = jnp.pad(pred_b2, (0, G - pred_b2.shape[0])).reshape(1, -1)

    out = _pool_pred(lh, ph, lb3, pb3, W1a, W1b,
                     pred_b1.reshape(1, -1), W2p, b2p)
    return out[:, :1]
